# Initial kernel scaffold; baseline (speedup 1.0000x reference)
import jax
import jax.numpy as jnp
from jax.experimental import pallas as pl


def kernel(D_feats, D_edge_index, D_graph_ids, T_feats, T_edge_index, T_edge_weight, T_seq, W_D, W_T1, W_T, fc1_W, fc1_b, fc2_W, fc2_b, p1_W, p1_b, p2_W, p2_b, p3_W, p3_b, fc_W, fc_b):
    raise NotImplementedError("write your pallas kernel here")


# speedup vs baseline: 1.0548x; 1.0548x over previous
import functools

import jax
import jax.numpy as jnp
from jax.experimental import pallas as pl
from jax.experimental.pallas import tpu as pltpu

B = 128
ND = 8192
NT = 65536
L = 512
D_FEAT = 128
T_IN = 64
HID = 256


def _normalize(x, eps=1e-12):
    n = jnp.sqrt(jnp.sum(x * x, axis=-1, keepdims=True))
    return x / jnp.maximum(n, eps)


BF = 8


def _fusion_body(t_seq_ref, t_feats_ref, t_agg_ref, w_t_ref, fc1w_ref, fc1b_ref,
                 w_t1_ref, fc2w_ref, fc2b_ref, op_ref):
    ts = t_seq_ref[...].reshape(BF * L, T_IN)
    tx = jnp.dot(ts, w_t_ref[...], preferred_element_type=jnp.float32)
    x1 = jnp.dot(tx, fc1w_ref[...], preferred_element_type=jnp.float32) + fc1b_ref[...]
    tf = (t_feats_ref[...] + t_agg_ref[...]).reshape(BF * L, D_FEAT)
    tx1 = jax.nn.relu(jnp.dot(tf, w_t1_ref[...], preferred_element_type=jnp.float32))
    x2 = jnp.dot(tx1, fc2w_ref[...], preferred_element_type=jnp.float32) + fc2b_ref[...]
    xco = (_normalize(x1) + _normalize(x2)).reshape(BF, L, HID)
    op_ref[...] = jnp.max(xco, axis=1)


def _fusion(T_seq, T_feats, t_agg, W_T, fc1_W, fc1_b, W_T1, fc2_W, fc2_b):
    grid = (B // BF,)
    return pl.pallas_call(
        _fusion_body,
        grid=grid,
        in_specs=[
            pl.BlockSpec((BF, L, T_IN), lambda b: (b, 0, 0)),
            pl.BlockSpec((BF, L, D_FEAT), lambda b: (b, 0, 0)),
            pl.BlockSpec((BF, L, D_FEAT), lambda b: (b, 0, 0)),
            pl.BlockSpec((T_IN, 128), lambda b: (0, 0)),
            pl.BlockSpec((128, HID), lambda b: (0, 0)),
            pl.BlockSpec((1, HID), lambda b: (0, 0)),
            pl.BlockSpec((D_FEAT, 128), lambda b: (0, 0)),
            pl.BlockSpec((128, HID), lambda b: (0, 0)),
            pl.BlockSpec((1, HID), lambda b: (0, 0)),
        ],
        out_specs=pl.BlockSpec((BF, HID), lambda b: (b, 0)),
        out_shape=jax.ShapeDtypeStruct((B, HID), jnp.float32),
    )(T_seq, T_feats, t_agg, W_T, fc1_W, fc1_b.reshape(1, HID),
      W_T1, fc2_W, fc2_b.reshape(1, HID))


def _head_body(d_feats_ref, d_agg_ref, ids_ref, wd_ref, op_ref,
               p1w_ref, p1b_ref, p2w_ref, p2b_ref, p3w_ref, p3b_ref,
               fcw_ref, fcb_ref, out_ref):
    hD = jax.nn.relu(jnp.dot(d_feats_ref[...] + d_agg_ref[...], wd_ref[...],
                             preferred_element_type=jnp.float32))
    ids = ids_ref[...]
    seg = jax.lax.broadcasted_iota(jnp.int32, (B, ND), 0)
    mask = (seg == ids).astype(jnp.float32)
    g_sum = jnp.dot(mask, hD, preferred_element_type=jnp.float32)
    g_cnt = jnp.sum(mask, axis=1, keepdims=True)
    d_x = g_sum / jnp.maximum(g_cnt, 1.0)
    dt = jnp.concatenate([d_x, op_ref[...]], axis=-1)
    h = jax.nn.relu(jnp.dot(dt, p1w_ref[...], preferred_element_type=jnp.float32) + p1b_ref[...])
    h = jax.nn.relu(jnp.dot(h, p2w_ref[...], preferred_element_type=jnp.float32) + p2b_ref[...])
    h = jnp.dot(h, p3w_ref[...], preferred_element_type=jnp.float32) + p3b_ref[...]
    out_ref[...] = jnp.dot(h, fcw_ref[...], preferred_element_type=jnp.float32) + fcb_ref[...]


def _head(D_feats, d_agg, ids_i32, W_D, op, p1_W, p1_b, p2_W, p2_b, p3_W, p3_b, fc_W, fc_b):
    return pl.pallas_call(
        _head_body,
        out_shape=jax.ShapeDtypeStruct((B, 2), jnp.float32),
    )(D_feats, d_agg, ids_i32.reshape(1, ND), W_D, op,
      p1_W, p1_b.reshape(1, -1), p2_W, p2_b.reshape(1, -1),
      p3_W, p3_b.reshape(1, -1), fc_W, fc_b.reshape(1, -1))


def kernel(D_feats, D_edge_index, D_graph_ids, T_feats, T_edge_index, T_edge_weight,
           T_seq, W_D, W_T1, W_T, fc1_W, fc1_b, fc2_W, fc2_b,
           p1_W, p1_b, p2_W, p2_b, p3_W, p3_b, fc_W, fc_b):
    d_src = D_edge_index[0].astype(jnp.int32)
    d_dst = D_edge_index[1].astype(jnp.int32)
    t_src = T_edge_index[0].astype(jnp.int32)
    t_dst = T_edge_index[1].astype(jnp.int32)
    ids = D_graph_ids.astype(jnp.int32)

    d_agg = jax.ops.segment_sum(jnp.take(D_feats, d_src, axis=0), d_dst, num_segments=ND)
    msgs = jnp.take(T_feats, t_src, axis=0) * T_edge_weight[:, None]
    t_agg = jax.ops.segment_sum(msgs, t_dst, num_segments=NT)

    op = _fusion(T_seq, T_feats.reshape(B, L, D_FEAT), t_agg.reshape(B, L, D_FEAT),
                 W_T, fc1_W, fc1_b, W_T1, fc2_W, fc2_b)
    out = _head(D_feats, d_agg, ids, W_D, op,
                p1_W, p1_b, p2_W, p2_b, p3_W, p3_b, fc_W, fc_b)
    return out
